# baseline (speedup 1.0000x reference)
import functools

import jax
import jax.numpy as jnp
from jax import lax
from jax.experimental import pallas as pl
from jax.experimental.pallas import tpu as pltpu
from jax.experimental.pallas import tpu_sc as plsc

Z_DIM = 128
EMBED_SIZE = 128
BATCH = 16384

NUM_CORES = 2
NUM_SUBCORES = 16
LANES = 16
NUM_WORKERS = NUM_CORES * NUM_SUBCORES
ROWS_PER_WORKER = BATCH // NUM_WORKERS
CHUNK = 128
CHUNKS_PER_WORKER = ROWS_PER_WORKER // CHUNK


def _vrsqrt(x):
    i = lax.bitcast_convert_type(x, jnp.int32)
    i = jnp.int32(0x5F3759DF) - lax.shift_right_arithmetic(i, 1)
    y = lax.bitcast_convert_type(i, jnp.float32)
    xhalf = x * 0.5
    for _ in range(3):
        y = y * (1.5 - xhalf * y * y)
    return y


def _body(z_hbm, y_hbm, emb_hbm, out_hbm, idx_v, rows_v, zbuf_v, sem):
    wid = lax.axis_index("s") * NUM_CORES + lax.axis_index("c")
    pltpu.sync_copy(y_hbm.at[pl.ds(wid * CHUNKS_PER_WORKER, CHUNKS_PER_WORKER)],
                    idx_v)

    for j in range(CHUNKS_PER_WORKER):
        base = wid * ROWS_PER_WORKER + j * CHUNK
        pltpu.async_copy(emb_hbm.at[idx_v.at[j]], rows_v, sem).wait()

        pltpu.sync_copy(z_hbm.at[pl.ds(base, CHUNK)], zbuf_v)
        pltpu.sync_copy(zbuf_v, out_hbm.at[pl.ds(base, CHUNK), pl.ds(0, Z_DIM)])

        def row_body(r, carry):
            vs = []
            acc = None
            for k in range(EMBED_SIZE // LANES):
                v = rows_v[r, pl.ds(k * LANES, LANES)]
                vs.append(v)
                acc = v * v if acc is None else acc + v * v
            for sh in (8, 4, 2, 1):
                perm = jnp.arange(LANES, dtype=jnp.int32) ^ sh
                acc = acc + acc.at[perm].get(mode="promise_in_bounds")
            rinv = _vrsqrt(acc)
            for k in range(EMBED_SIZE // LANES):
                rows_v[r, pl.ds(k * LANES, LANES)] = vs[k] * rinv
            return carry

        lax.fori_loop(0, CHUNK, row_body, 0)

        pltpu.sync_copy(rows_v,
                        out_hbm.at[pl.ds(base, CHUNK), pl.ds(Z_DIM, EMBED_SIZE)])


@jax.jit
def kernel(z, y, embedding):
    y2 = y.reshape(BATCH // CHUNK, CHUNK)
    mesh = plsc.VectorSubcoreMesh(core_axis_name="c", subcore_axis_name="s",
                                  num_cores=NUM_CORES, num_subcores=NUM_SUBCORES)
    run = pl.kernel(
        _body,
        out_type=jax.ShapeDtypeStruct((BATCH, Z_DIM + EMBED_SIZE), jnp.float32),
        mesh=mesh,
        scratch_types=[
            pltpu.VMEM((CHUNKS_PER_WORKER, CHUNK), jnp.int32),
            pltpu.VMEM((CHUNK, EMBED_SIZE), jnp.float32),
            pltpu.VMEM((CHUNK, Z_DIM), jnp.float32),
            pltpu.SemaphoreType.DMA,
        ],
    )
    return run(z, y2, embedding)
